# baseline (speedup 1.0000x reference)
import jax
import jax.numpy as jnp
from jax import lax
from jax.experimental import pallas as pl
from jax.experimental.pallas import tpu as pltpu

_E = 8
_R = 16
_SCALING = 2.0
_TN = 256


def _moe_kernel(x_ref, wb_ref, wr_ref, aall_ref, ball_ref, expand_ref, bbase_ref,
                out_ref, wbbf_ref):
    @pl.when(pl.program_id(0) == 0)
    def _cast_base():
        wbbf_ref[...] = wb_ref[...].astype(jnp.bfloat16)

    x = x_ref[...].astype(jnp.bfloat16)
    logits = lax.dot_general(
        x, wr_ref[...], (((1,), (1,)), ((), ())),
        preferred_element_type=jnp.float32)
    m1 = jnp.max(logits, axis=1, keepdims=True)
    rest = jnp.where(logits < m1, logits, -jnp.inf)
    m2 = jnp.max(rest, axis=1, keepdims=True)
    w = jnp.where(logits >= m2, jnp.exp(logits - m1), 0.0)
    gate = w / jnp.sum(w, axis=1, keepdims=True)
    xa = lax.dot_general(
        x, aall_ref[...], (((1,), (1,)), ((), ())),
        preferred_element_type=jnp.float32)
    scale = lax.dot_general(
        gate, expand_ref[...], (((1,), (0,)), ((), ())),
        preferred_element_type=jnp.float32)
    xa = (xa * scale).astype(jnp.bfloat16)
    base = lax.dot_general(
        x, wbbf_ref[...], (((1,), (1,)), ((), ())),
        preferred_element_type=jnp.float32)
    lora = lax.dot_general(
        xa, ball_ref[...], (((1,), (1,)), ((), ())),
        preferred_element_type=jnp.float32)
    out_ref[...] = base + lora + bbase_ref[...]


def kernel(x, W_base, b_base, W_router, A, B):
    bs, seq, d = x.shape
    out_dim = W_base.shape[0]
    n = bs * seq
    x2 = x.reshape(n, d)
    wr = W_router.astype(jnp.bfloat16)
    a_all = A.reshape(_E * _R, d).astype(jnp.bfloat16)
    b_all = jnp.transpose(B, (1, 0, 2)).reshape(out_dim, _E * _R)
    b_all = b_all.astype(jnp.bfloat16)
    bias = b_base.reshape(1, out_dim)
    expand = jnp.kron(jnp.eye(_E, dtype=jnp.float32),
                      jnp.ones((1, _R), jnp.float32)) * _SCALING

    grid = (n // _TN,)
    out = pl.pallas_call(
        _moe_kernel,
        grid=grid,
        in_specs=[
            pl.BlockSpec((_TN, d), lambda i: (i, 0)),
            pl.BlockSpec((out_dim, d), lambda i: (0, 0)),
            pl.BlockSpec((_E, d), lambda i: (0, 0)),
            pl.BlockSpec((_E * _R, d), lambda i: (0, 0)),
            pl.BlockSpec((out_dim, _E * _R), lambda i: (0, 0)),
            pl.BlockSpec((_E, _E * _R), lambda i: (0, 0)),
            pl.BlockSpec((1, out_dim), lambda i: (0, 0)),
        ],
        out_specs=pl.BlockSpec((_TN, out_dim), lambda i: (i, 0)),
        out_shape=jax.ShapeDtypeStruct((n, out_dim), jnp.float32),
        scratch_shapes=[pltpu.VMEM((out_dim, d), jnp.bfloat16)],
        compiler_params=pltpu.CompilerParams(
            dimension_semantics=("arbitrary",),
        ),
    )(x2, W_base, wr, a_all, b_all, expand, bias)
    return out.reshape(bs, seq, out_dim)


# speedup vs baseline: 1.1348x; 1.1348x over previous
import jax
import jax.numpy as jnp
from jax import lax
from jax.experimental import pallas as pl
from jax.experimental.pallas import tpu as pltpu

_E = 8
_R = 16
_SCALING = 2.0
_TN = 512


def _moe_kernel(x_ref, wb_ref, wr_ref, aall_ref, ball_ref, expand_ref, bbase_ref,
                out_ref):
    x = x_ref[...]
    logits = lax.dot_general(
        x, wr_ref[...], (((1,), (1,)), ((), ())),
        preferred_element_type=jnp.float32)
    m1 = jnp.max(logits, axis=1, keepdims=True)
    rest = jnp.where(logits < m1, logits, -jnp.inf)
    m2 = jnp.max(rest, axis=1, keepdims=True)
    w = jnp.where(logits >= m2, jnp.exp(logits - m1), 0.0)
    gate = w / jnp.sum(w, axis=1, keepdims=True)
    xa = lax.dot_general(
        x, aall_ref[...], (((1,), (1,)), ((), ())),
        preferred_element_type=jnp.float32)
    scale = lax.dot_general(
        gate, expand_ref[...], (((1,), (0,)), ((), ())),
        preferred_element_type=jnp.float32)
    xa = xa * scale
    base = lax.dot_general(
        x, wb_ref[...], (((1,), (1,)), ((), ())),
        preferred_element_type=jnp.float32)
    lora = lax.dot_general(
        xa, ball_ref[...], (((1,), (1,)), ((), ())),
        preferred_element_type=jnp.float32)
    out_ref[...] = base + lora + bbase_ref[...]


def kernel(x, W_base, b_base, W_router, A, B):
    bs, seq, d = x.shape
    out_dim = W_base.shape[0]
    n = bs * seq
    x2 = x.reshape(n, d)
    wb = W_base
    wr = W_router
    a_all = A.reshape(_E * _R, d)
    b_all = jnp.transpose(B, (1, 0, 2)).reshape(out_dim, _E * _R)
    bias = b_base.reshape(1, out_dim)
    expand = jnp.kron(jnp.eye(_E, dtype=jnp.float32),
                      jnp.ones((1, _R), jnp.float32)) * _SCALING

    grid = (n // _TN,)
    out = pl.pallas_call(
        _moe_kernel,
        grid=grid,
        in_specs=[
            pl.BlockSpec((_TN, d), lambda i: (i, 0)),
            pl.BlockSpec((out_dim, d), lambda i: (0, 0)),
            pl.BlockSpec((_E, d), lambda i: (0, 0)),
            pl.BlockSpec((_E * _R, d), lambda i: (0, 0)),
            pl.BlockSpec((out_dim, _E * _R), lambda i: (0, 0)),
            pl.BlockSpec((_E, _E * _R), lambda i: (0, 0)),
            pl.BlockSpec((1, out_dim), lambda i: (0, 0)),
        ],
        out_specs=pl.BlockSpec((_TN, out_dim), lambda i: (i, 0)),
        out_shape=jax.ShapeDtypeStruct((n, out_dim), jnp.float32),
        compiler_params=pltpu.CompilerParams(
            dimension_semantics=("arbitrary",),
        ),
    )(x2, wb, wr, a_all, b_all, expand, bias)
    return out.reshape(bs, seq, out_dim)


# speedup vs baseline: 1.1372x; 1.0021x over previous
import jax
import jax.numpy as jnp
from jax import lax
from jax.experimental import pallas as pl
from jax.experimental.pallas import tpu as pltpu

_E = 8
_R = 16
_SCALING = 2.0
_TN = 1024


def _moe_kernel(x_ref, wb_ref, wr_ref, aall_ref, ball_ref, expand_ref, bbase_ref,
                out_ref):
    x = x_ref[...]
    logits = lax.dot_general(
        x, wr_ref[...], (((1,), (1,)), ((), ())),
        preferred_element_type=jnp.float32)
    m1 = jnp.max(logits, axis=1, keepdims=True)
    rest = jnp.where(logits < m1, logits, -jnp.inf)
    m2 = jnp.max(rest, axis=1, keepdims=True)
    w = jnp.where(logits >= m2, jnp.exp(logits - m1), 0.0)
    gate = w / jnp.sum(w, axis=1, keepdims=True)
    xa = lax.dot_general(
        x, aall_ref[...], (((1,), (1,)), ((), ())),
        preferred_element_type=jnp.float32)
    scale = lax.dot_general(
        gate, expand_ref[...], (((1,), (0,)), ((), ())),
        preferred_element_type=jnp.float32)
    xa = xa * scale
    base = lax.dot_general(
        x, wb_ref[...], (((1,), (1,)), ((), ())),
        preferred_element_type=jnp.float32)
    lora = lax.dot_general(
        xa, ball_ref[...], (((1,), (1,)), ((), ())),
        preferred_element_type=jnp.float32)
    out_ref[...] = base + lora + bbase_ref[...]


def kernel(x, W_base, b_base, W_router, A, B):
    bs, seq, d = x.shape
    out_dim = W_base.shape[0]
    n = bs * seq
    x2 = x.reshape(n, d)
    wb = W_base
    wr = W_router
    a_all = A.reshape(_E * _R, d)
    b_all = jnp.transpose(B, (1, 0, 2)).reshape(out_dim, _E * _R)
    bias = b_base.reshape(1, out_dim)
    expand = jnp.kron(jnp.eye(_E, dtype=jnp.float32),
                      jnp.ones((1, _R), jnp.float32)) * _SCALING

    grid = (n // _TN,)
    out = pl.pallas_call(
        _moe_kernel,
        grid=grid,
        in_specs=[
            pl.BlockSpec((_TN, d), lambda i: (i, 0)),
            pl.BlockSpec((out_dim, d), lambda i: (0, 0)),
            pl.BlockSpec((_E, d), lambda i: (0, 0)),
            pl.BlockSpec((_E * _R, d), lambda i: (0, 0)),
            pl.BlockSpec((out_dim, _E * _R), lambda i: (0, 0)),
            pl.BlockSpec((_E, _E * _R), lambda i: (0, 0)),
            pl.BlockSpec((1, out_dim), lambda i: (0, 0)),
        ],
        out_specs=pl.BlockSpec((_TN, out_dim), lambda i: (i, 0)),
        out_shape=jax.ShapeDtypeStruct((n, out_dim), jnp.float32),
        compiler_params=pltpu.CompilerParams(
            dimension_semantics=("arbitrary",),
        ),
    )(x2, wb, wr, a_all, b_all, expand, bias)
    return out.reshape(bs, seq, out_dim)
